# Initial kernel scaffold; baseline (speedup 1.0000x reference)
import jax
import jax.numpy as jnp
from jax.experimental import pallas as pl


def kernel(x, edge_index, W1, b1, W2, b2, Wa, Wm1, bm1, Wm2, bm2, Wm3, bm3):
    raise NotImplementedError("write your pallas kernel here")


# speedup vs baseline: 4.9992x; 4.9992x over previous
import functools

import jax
import jax.numpy as jnp
from jax import lax
from jax.experimental import pallas as pl
from jax.experimental.pallas import tpu as pltpu
from jax.experimental.pallas import tpu_sc as plsc

N = 10000
E = 320000
D_IN = 128
D1 = 80
D2 = 40
D2P = 48

NC = 2
NS = 16
NW = NC * NS
EPW = E // NW
CH = 80
NCHUNK = EPW // CH
NPAD = 10240
RPT = NPAD // NS

_MESH = dict(core_axis_name="c", subcore_axis_name="s",
             num_cores=NC, num_subcores=NS)


@functools.cache
def _get_sc_degrees():
    return functools.partial(
        pl.kernel,
        out_type=jax.ShapeDtypeStruct((NC, 2, NPAD), jnp.float32),
        mesh=plsc.VectorSubcoreMesh(**_MESH),
        scratch_types=[
            pltpu.VMEM((CH,), jnp.int32),
            pltpu.VMEM((CH,), jnp.float32),
            pltpu.VMEM_SHARED((NPAD,), jnp.float32),
            pltpu.VMEM_SHARED((NPAD,), jnp.float32),
        ],
    )(_sc_degrees_body)


def _sc_degrees_body(src_hbm, dst_hbm, zdeg_hbm, out_hbm, idx_v, ones_v,
                     acc_out, acc_in):
    c = lax.axis_index("c")
    s = lax.axis_index("s")
    wid = c * jnp.int32(NS) + s
    one = jnp.ones((16,), jnp.float32)
    for j in range(CH // 16):
        ones_v[pl.ds(j * 16, 16)] = one
    zoff = s * jnp.int32(RPT)
    pltpu.sync_copy(zdeg_hbm.at[pl.ds(zoff, RPT)],
                    acc_out.at[pl.ds(zoff, RPT)])
    pltpu.sync_copy(zdeg_hbm.at[pl.ds(zoff, RPT)],
                    acc_in.at[pl.ds(zoff, RPT)])
    plsc.subcore_barrier()
    base = wid * jnp.int32(EPW)

    def body(i, carry):
        off = base + i * jnp.int32(CH)
        pltpu.sync_copy(src_hbm.at[pl.ds(off, CH)], idx_v)
        pltpu.sync_copy(ones_v, acc_out.at[idx_v], add=True)
        pltpu.sync_copy(dst_hbm.at[pl.ds(off, CH)], idx_v)
        pltpu.sync_copy(ones_v, acc_in.at[idx_v], add=True)
        return carry

    lax.fori_loop(jnp.int32(0), jnp.int32(NCHUNK), body, jnp.int32(0))
    plsc.subcore_barrier()
    pltpu.sync_copy(acc_out.at[pl.ds(zoff, RPT)],
                    out_hbm.at[c, jnp.int32(0), pl.ds(zoff, RPT)])
    pltpu.sync_copy(acc_in.at[pl.ds(zoff, RPT)],
                    out_hbm.at[c, jnp.int32(1), pl.ds(zoff, RPT)])


@functools.cache
def _make_sc_aggregate(d):
    @functools.partial(
        pl.kernel,
        out_type=jax.ShapeDtypeStruct((NC, NPAD, d), jnp.float32),
        mesh=plsc.VectorSubcoreMesh(**_MESH),
        scratch_types=[
            pltpu.VMEM((CH,), jnp.int32),
            pltpu.VMEM((CH,), jnp.int32),
            pltpu.VMEM((CH, d), jnp.float32),
            pltpu.VMEM_SHARED((NPAD, d), jnp.float32),
            pltpu.SemaphoreType.DMA,
        ],
        compiler_params=pltpu.CompilerParams(use_tc_tiling_on_sc=False),
    )
    def agg(v_hbm, src_hbm, dst_hbm, zeros_hbm, out_hbm,
            sidx, didx, rows, acc, sem):
        c = lax.axis_index("c")
        s = lax.axis_index("s")
        wid = c * jnp.int32(NS) + s
        zoff = s * jnp.int32(RPT)
        pltpu.sync_copy(zeros_hbm.at[pl.ds(zoff, RPT)],
                        acc.at[pl.ds(zoff, RPT)])
        plsc.subcore_barrier()
        base = wid * jnp.int32(EPW)

        def body(i, carry):
            off = base + i * jnp.int32(CH)
            pltpu.sync_copy(src_hbm.at[pl.ds(off, CH)], sidx)
            pltpu.sync_copy(dst_hbm.at[pl.ds(off, CH)], didx)
            pltpu.async_copy(v_hbm.at[sidx], rows, sem).wait()
            pltpu.sync_copy(rows, acc.at[didx], add=True)
            return carry

        lax.fori_loop(jnp.int32(0), jnp.int32(NCHUNK), body, jnp.int32(0))
        plsc.subcore_barrier()
        pltpu.sync_copy(acc.at[pl.ds(zoff, RPT)],
                        out_hbm.at[c, pl.ds(zoff, RPT)])

    return agg


def _tc_mm1_body(x_ref, w1_ref, o_ref):
    o_ref[...] = jnp.dot(x_ref[...], w1_ref[...],
                         preferred_element_type=jnp.float32)


def _tc_scale_body(degp_ref, u1_ref, v1_ref, rsqi_ref, rsqo_ref):
    dp = degp_ref[...]
    deg_out = dp[0, 0, :N] + dp[1, 0, :N]
    deg_in = dp[0, 1, :N] + dp[1, 1, :N]
    rsq_out = lax.rsqrt(jnp.maximum(deg_out, 1.0))[:, None]
    rsq_in = lax.rsqrt(jnp.maximum(deg_in, 1.0))[:, None]
    v1_ref[...] = u1_ref[...] * rsq_out
    rsqi_ref[...] = rsq_in
    rsqo_ref[...] = rsq_out


def _tc_layer2_body(aggp_ref, rsqi_ref, rsqo_ref, b1_ref, w2p_ref, v2_ref):
    a = aggp_ref[0, :N, :] + aggp_ref[1, :N, :]
    h1 = jnp.maximum(a * rsqi_ref[...] + b1_ref[...][None, :], 0.0)
    v2_ref[...] = jnp.dot(h1 * rsqo_ref[...], w2p_ref[...],
                          preferred_element_type=jnp.float32)


def _tc_head_body(aggp_ref, rsqi_ref, b2_ref, wa_ref, wm1_ref, bm1_ref,
                  wm2_ref, bm2_ref, wm3_ref, bm3_ref, o_ref):
    a = aggp_ref[0, :N, :D2] + aggp_ref[1, :N, :D2]
    h2 = jnp.maximum(a * rsqi_ref[...] + b2_ref[...][None, :], 0.0)
    m = jnp.mean(h2, axis=0, keepdims=True)
    tg = jnp.tanh(jnp.dot(m, wa_ref[...],
                          preferred_element_type=jnp.float32))
    scores = jax.nn.sigmoid(jnp.sum(h2 * tg, axis=1, keepdims=True))
    rep = jnp.sum(h2 * scores, axis=0, keepdims=True)
    z = jnp.maximum(jnp.dot(rep, wm1_ref[...],
                            preferred_element_type=jnp.float32)
                    + bm1_ref[...][None, :], 0.0)
    z = jnp.maximum(jnp.dot(z, wm2_ref[...],
                            preferred_element_type=jnp.float32)
                    + bm2_ref[...][None, :], 0.0)
    o_ref[...] = (jnp.dot(z, wm3_ref[...],
                          preferred_element_type=jnp.float32)
                  + bm3_ref[...][None, :])


def _tc_call(body, out_shape, *args):
    return pl.pallas_call(body, out_shape=out_shape)(*args)


def kernel(x, edge_index, W1, b1, W2, b2, Wa, Wm1, bm1, Wm2, bm2, Wm3, bm3):
    src = edge_index[0].astype(jnp.int32)
    dst = edge_index[1].astype(jnp.int32)
    zdeg = jnp.zeros((NPAD,), jnp.float32)
    z1 = jnp.zeros((NPAD, D1), jnp.float32)
    z2 = jnp.zeros((NPAD, D2P), jnp.float32)
    W2p = jnp.pad(W2, ((0, 0), (0, D2P - D2)))

    degp = _get_sc_degrees()(src, dst, zdeg)
    u1 = _tc_call(_tc_mm1_body,
                  jax.ShapeDtypeStruct((N, D1), jnp.float32), x, W1)
    v1, rsq_in, rsq_out = _tc_call(
        _tc_scale_body,
        (jax.ShapeDtypeStruct((N, D1), jnp.float32),
         jax.ShapeDtypeStruct((N, 1), jnp.float32),
         jax.ShapeDtypeStruct((N, 1), jnp.float32)),
        degp, u1)
    agg1p = _make_sc_aggregate(D1)(v1, src, dst, z1)
    v2 = _tc_call(_tc_layer2_body,
                  jax.ShapeDtypeStruct((N, D2P), jnp.float32),
                  agg1p, rsq_in, rsq_out, b1, W2p)
    agg2p = _make_sc_aggregate(D2P)(v2, src, dst, z2)
    out = _tc_call(_tc_head_body,
                   jax.ShapeDtypeStruct((1, 1), jnp.float32),
                   agg2p, rsq_in, b2, Wa, Wm1, bm1, Wm2, bm2, Wm3, bm3)
    return out


# speedup vs baseline: 10.6857x; 2.1375x over previous
import functools

import jax
import jax.numpy as jnp
from jax import lax
from jax.experimental import pallas as pl
from jax.experimental.pallas import tpu as pltpu
from jax.experimental.pallas import tpu_sc as plsc

N = 10000
E = 320000
D_IN = 128
D1 = 80
D2 = 40
D2P = 48

NC = 2
NS = 16
NW = NC * NS
EPW = E // NW
CH = 40
NCHUNK = EPW // CH
NBUF = 10
PD = 5
NG = NCHUNK // NBUF
NPAD = 10240
RPT = NPAD // NS

_MESH = dict(core_axis_name="c", subcore_axis_name="s",
             num_cores=NC, num_subcores=NS)


_DCH = 80
_DNCH = EPW // _DCH


@functools.cache
def _get_sc_degrees():
    return functools.partial(
        pl.kernel,
        out_type=jax.ShapeDtypeStruct((NC, 2, NPAD), jnp.float32),
        mesh=plsc.VectorSubcoreMesh(**_MESH),
        scratch_types=[
            pltpu.VMEM((_DCH,), jnp.int32),
            pltpu.VMEM((_DCH,), jnp.float32),
            pltpu.VMEM_SHARED((NPAD,), jnp.float32),
            pltpu.VMEM_SHARED((NPAD,), jnp.float32),
        ],
    )(_sc_degrees_body)


def _sc_degrees_body(src_hbm, dst_hbm, zdeg_hbm, out_hbm, idx_v, ones_v,
                     acc_out, acc_in):
    c = lax.axis_index("c")
    s = lax.axis_index("s")
    wid = c * jnp.int32(NS) + s
    one = jnp.ones((16,), jnp.float32)
    for j in range(_DCH // 16):
        ones_v[pl.ds(j * 16, 16)] = one
    zoff = s * jnp.int32(RPT)
    pltpu.sync_copy(zdeg_hbm.at[pl.ds(zoff, RPT)],
                    acc_out.at[pl.ds(zoff, RPT)])
    pltpu.sync_copy(zdeg_hbm.at[pl.ds(zoff, RPT)],
                    acc_in.at[pl.ds(zoff, RPT)])
    plsc.subcore_barrier()
    base = wid * jnp.int32(EPW)

    def body(i, carry):
        off = base + i * jnp.int32(_DCH)
        pltpu.sync_copy(src_hbm.at[pl.ds(off, _DCH)], idx_v)
        pltpu.sync_copy(ones_v, acc_out.at[idx_v], add=True)
        pltpu.sync_copy(dst_hbm.at[pl.ds(off, _DCH)], idx_v)
        pltpu.sync_copy(ones_v, acc_in.at[idx_v], add=True)
        return carry

    lax.fori_loop(jnp.int32(0), jnp.int32(_DNCH), body, jnp.int32(0))
    plsc.subcore_barrier()
    pltpu.sync_copy(acc_out.at[pl.ds(zoff, RPT)],
                    out_hbm.at[c, jnp.int32(0), pl.ds(zoff, RPT)])
    pltpu.sync_copy(acc_in.at[pl.ds(zoff, RPT)],
                    out_hbm.at[c, jnp.int32(1), pl.ds(zoff, RPT)])


@functools.cache
def _make_sc_aggregate(d):
    @functools.partial(
        pl.kernel,
        out_type=jax.ShapeDtypeStruct((NC, NPAD, d), jnp.float32),
        mesh=plsc.VectorSubcoreMesh(**_MESH),
        scratch_types=(
            [pltpu.VMEM((NCHUNK, CH), jnp.int32)]
            + [pltpu.VMEM((CH, d), jnp.float32)] * NBUF
            + [pltpu.VMEM((CH,), jnp.int32)] * NBUF
            + [pltpu.SemaphoreType.DMA] * (2 * NBUF)
            + [pltpu.VMEM_SHARED((NPAD, d), jnp.float32)]
        ),
        compiler_params=pltpu.CompilerParams(use_tc_tiling_on_sc=False),
    )
    def agg(v_hbm, src_hbm, dst_hbm, zeros_hbm, out_hbm, *scratch):
        sidx = scratch[0]
        rows = scratch[1:1 + NBUF]
        didxb = scratch[1 + NBUF:1 + 2 * NBUF]
        gsem = scratch[1 + 2 * NBUF:1 + 3 * NBUF]
        dsem = scratch[1 + 3 * NBUF:1 + 4 * NBUF]
        acc = scratch[1 + 4 * NBUF]
        c = lax.axis_index("c")
        s = lax.axis_index("s")
        wid = c * jnp.int32(NS) + s
        zoff = s * jnp.int32(RPT)
        pltpu.sync_copy(src_hbm.at[wid], sidx)
        pltpu.sync_copy(zeros_hbm.at[pl.ds(zoff, RPT)],
                        acc.at[pl.ds(zoff, RPT)])
        plsc.subcore_barrier()

        def wait_fetch(b, chunk):
            pltpu.make_async_copy(v_hbm.at[sidx.at[chunk]], rows[b],
                                  gsem[b]).wait()
            pltpu.make_async_copy(dst_hbm.at[wid, chunk], didxb[b],
                                  dsem[b]).wait()

        def fetch(chunk, b):
            pltpu.async_copy(v_hbm.at[sidx.at[chunk]], rows[b], gsem[b])
            pltpu.async_copy(dst_hbm.at[wid, chunk], didxb[b], dsem[b])

        for b in range(PD):
            fetch(jnp.int32(b), b)

        def group(g, carry):
            for b in range(NBUF):
                i = g * jnp.int32(NBUF) + jnp.int32(b)
                j = i + jnp.int32(PD)
                bj = (b + PD) % NBUF
                wait_fetch(b, i)
                if b < PD:
                    fetch(j, bj)
                else:
                    @pl.when(g < jnp.int32(NG - 1))
                    def _():
                        fetch(j, bj)
                pltpu.sync_copy(rows[b], acc.at[didxb[b]], add=True)
            return carry

        lax.fori_loop(jnp.int32(0), jnp.int32(NG), group, jnp.int32(0))
        plsc.subcore_barrier()
        pltpu.sync_copy(acc.at[pl.ds(zoff, RPT)],
                        out_hbm.at[c, pl.ds(zoff, RPT)])

    return agg


def _tc_mm1_body(x_ref, w1_ref, o_ref):
    o_ref[...] = jnp.dot(x_ref[...], w1_ref[...],
                         preferred_element_type=jnp.float32)


def _tc_scale_body(degp_ref, u1_ref, v1_ref, rsqi_ref, rsqo_ref):
    dp = degp_ref[...]
    deg_out = dp[0, 0, :N] + dp[1, 0, :N]
    deg_in = dp[0, 1, :N] + dp[1, 1, :N]
    rsq_out = lax.rsqrt(jnp.maximum(deg_out, 1.0))[:, None]
    rsq_in = lax.rsqrt(jnp.maximum(deg_in, 1.0))[:, None]
    v1_ref[...] = u1_ref[...] * rsq_out
    rsqi_ref[...] = rsq_in
    rsqo_ref[...] = rsq_out


def _tc_layer2_body(aggp_ref, rsqi_ref, rsqo_ref, b1_ref, w2p_ref, v2_ref):
    a = aggp_ref[0, :N, :] + aggp_ref[1, :N, :]
    h1 = jnp.maximum(a * rsqi_ref[...] + b1_ref[...][None, :], 0.0)
    v2_ref[...] = jnp.dot(h1 * rsqo_ref[...], w2p_ref[...],
                          preferred_element_type=jnp.float32)


def _tc_head_body(aggp_ref, rsqi_ref, b2_ref, wa_ref, wm1_ref, bm1_ref,
                  wm2_ref, bm2_ref, wm3_ref, bm3_ref, o_ref):
    a = aggp_ref[0, :N, :D2] + aggp_ref[1, :N, :D2]
    h2 = jnp.maximum(a * rsqi_ref[...] + b2_ref[...][None, :], 0.0)
    m = jnp.mean(h2, axis=0, keepdims=True)
    tg = jnp.tanh(jnp.dot(m, wa_ref[...],
                          preferred_element_type=jnp.float32))
    scores = jax.nn.sigmoid(jnp.sum(h2 * tg, axis=1, keepdims=True))
    rep = jnp.sum(h2 * scores, axis=0, keepdims=True)
    z = jnp.maximum(jnp.dot(rep, wm1_ref[...],
                            preferred_element_type=jnp.float32)
                    + bm1_ref[...][None, :], 0.0)
    z = jnp.maximum(jnp.dot(z, wm2_ref[...],
                            preferred_element_type=jnp.float32)
                    + bm2_ref[...][None, :], 0.0)
    o_ref[...] = (jnp.dot(z, wm3_ref[...],
                          preferred_element_type=jnp.float32)
                  + bm3_ref[...][None, :])


def _tc_call(body, out_shape, *args):
    return pl.pallas_call(body, out_shape=out_shape)(*args)


def kernel(x, edge_index, W1, b1, W2, b2, Wa, Wm1, bm1, Wm2, bm2, Wm3, bm3):
    srcf = edge_index[0].astype(jnp.int32)
    dstf = edge_index[1].astype(jnp.int32)
    src = srcf.reshape(NW, NCHUNK, CH)
    dst = dstf.reshape(NW, NCHUNK, CH)
    zdeg = jnp.zeros((NPAD,), jnp.float32)
    z1 = jnp.zeros((NPAD, D1), jnp.float32)
    z2 = jnp.zeros((NPAD, D2P), jnp.float32)
    W2p = jnp.pad(W2, ((0, 0), (0, D2P - D2)))

    degp = _get_sc_degrees()(srcf, dstf, zdeg)
    u1 = _tc_call(_tc_mm1_body,
                  jax.ShapeDtypeStruct((N, D1), jnp.float32), x, W1)
    v1, rsq_in, rsq_out = _tc_call(
        _tc_scale_body,
        (jax.ShapeDtypeStruct((N, D1), jnp.float32),
         jax.ShapeDtypeStruct((N, 1), jnp.float32),
         jax.ShapeDtypeStruct((N, 1), jnp.float32)),
        degp, u1)
    agg1p = _make_sc_aggregate(D1)(v1, src, dst, z1)
    v2 = _tc_call(_tc_layer2_body,
                  jax.ShapeDtypeStruct((N, D2P), jnp.float32),
                  agg1p, rsq_in, rsq_out, b1, W2p)
    agg2p = _make_sc_aggregate(D2P)(v2, src, dst, z2)
    out = _tc_call(_tc_head_body,
                   jax.ShapeDtypeStruct((1, 1), jnp.float32),
                   agg2p, rsq_in, b2, Wa, Wm1, bm1, Wm2, bm2, Wm3, bm3)
    return out


# speedup vs baseline: 15.3381x; 1.4354x over previous
import functools

import jax
import jax.numpy as jnp
from jax import lax
from jax.experimental import pallas as pl
from jax.experimental.pallas import tpu as pltpu
from jax.experimental.pallas import tpu_sc as plsc

N = 10000
E = 320000
D_IN = 128
D1 = 80
D2 = 40
D2P = 48

NC = 2
NS = 16
NW = NC * NS
EPW = E // NW
CH = 40
NCHUNK = EPW // CH
NBUF = 10
PD = 5
NG = NCHUNK // NBUF
NPAD = 10240
RPT = NPAD // NS

_MESH = dict(core_axis_name="c", subcore_axis_name="s",
             num_cores=NC, num_subcores=NS)


_DCH = 80
_DNCH = EPW // _DCH
_DRING = 5


@functools.cache
def _get_sc_degrees():
    return functools.partial(
        pl.kernel,
        out_type=jax.ShapeDtypeStruct((NC, 2, NPAD), jnp.float32),
        mesh=plsc.VectorSubcoreMesh(**_MESH),
        scratch_types=(
            [pltpu.VMEM((_DCH,), jnp.float32)]
            + [pltpu.VMEM((_DCH,), jnp.int32)] * (2 * _DRING)
            + [pltpu.SemaphoreType.DMA] * (2 * _DRING + 2)
            + [pltpu.VMEM_SHARED((NPAD,), jnp.float32)] * 2
        ),
    )(_sc_degrees_body)


def _sc_degrees_body(src_hbm, dst_hbm, zdeg_hbm, out_hbm, *scratch):
    ones_v = scratch[0]
    sb = scratch[1:1 + _DRING]
    db = scratch[1 + _DRING:1 + 2 * _DRING]
    fs = scratch[1 + 2 * _DRING:1 + 3 * _DRING]
    fd = scratch[1 + 3 * _DRING:1 + 4 * _DRING]
    so, si = scratch[1 + 4 * _DRING], scratch[2 + 4 * _DRING]
    acc_out, acc_in = scratch[3 + 4 * _DRING], scratch[4 + 4 * _DRING]
    c = lax.axis_index("c")
    s = lax.axis_index("s")
    wid = c * jnp.int32(NS) + s
    one = jnp.ones((16,), jnp.float32)
    for j in range(_DCH // 16):
        ones_v[pl.ds(j * 16, 16)] = one
    zoff = s * jnp.int32(RPT)
    pltpu.sync_copy(zdeg_hbm.at[pl.ds(zoff, RPT)],
                    acc_out.at[pl.ds(zoff, RPT)])
    pltpu.sync_copy(zdeg_hbm.at[pl.ds(zoff, RPT)],
                    acc_in.at[pl.ds(zoff, RPT)])
    plsc.subcore_barrier()
    base = wid * jnp.int32(EPW)

    def fetch(chunk, b):
        off = base + chunk * jnp.int32(_DCH)
        pltpu.async_copy(src_hbm.at[pl.ds(off, _DCH)], sb[b], fs[b])
        pltpu.async_copy(dst_hbm.at[pl.ds(off, _DCH)], db[b], fd[b])

    for b in range(_DRING):
        fetch(jnp.int32(b), b)

    def group(g, carry):
        for b in range(_DRING):
            i = g * jnp.int32(_DRING) + jnp.int32(b)
            off = base + i * jnp.int32(_DCH)
            pltpu.make_async_copy(src_hbm.at[pl.ds(off, _DCH)], sb[b],
                                  fs[b]).wait()
            pltpu.make_async_copy(dst_hbm.at[pl.ds(off, _DCH)], db[b],
                                  fd[b]).wait()
            pltpu.sync_copy(ones_v, acc_out.at[sb[b]], add=True)
            pltpu.sync_copy(ones_v, acc_in.at[db[b]], add=True)
            j = i + jnp.int32(_DRING)

            @pl.when(j < jnp.int32(_DNCH))
            def _():
                fetch(j, b)
        return carry

    lax.fori_loop(jnp.int32(0), jnp.int32(_DNCH // _DRING), group,
                  jnp.int32(0))
    plsc.subcore_barrier()
    pltpu.sync_copy(acc_out.at[pl.ds(zoff, RPT)],
                    out_hbm.at[c, jnp.int32(0), pl.ds(zoff, RPT)])
    pltpu.sync_copy(acc_in.at[pl.ds(zoff, RPT)],
                    out_hbm.at[c, jnp.int32(1), pl.ds(zoff, RPT)])


@functools.cache
def _make_sc_aggregate(d):
    @functools.partial(
        pl.kernel,
        out_type=jax.ShapeDtypeStruct((NC, NPAD, d), jnp.float32),
        mesh=plsc.VectorSubcoreMesh(**_MESH),
        scratch_types=(
            [pltpu.VMEM((NCHUNK, CH), jnp.int32)]
            + [pltpu.VMEM((CH, d), jnp.float32)] * NBUF
            + [pltpu.VMEM((CH,), jnp.int32)] * NBUF
            + [pltpu.SemaphoreType.DMA] * (2 * NBUF)
            + [pltpu.VMEM_SHARED((NPAD, d), jnp.float32)]
        ),
        compiler_params=pltpu.CompilerParams(use_tc_tiling_on_sc=False),
    )
    def agg(v_hbm, src_hbm, dst_hbm, zeros_hbm, out_hbm, *scratch):
        sidx = scratch[0]
        rows = scratch[1:1 + NBUF]
        didxb = scratch[1 + NBUF:1 + 2 * NBUF]
        gsem = scratch[1 + 2 * NBUF:1 + 3 * NBUF]
        dsem = scratch[1 + 3 * NBUF:1 + 4 * NBUF]
        acc = scratch[1 + 4 * NBUF]
        c = lax.axis_index("c")
        s = lax.axis_index("s")
        wid = c * jnp.int32(NS) + s
        zoff = s * jnp.int32(RPT)
        pltpu.sync_copy(src_hbm.at[wid], sidx)
        pltpu.sync_copy(zeros_hbm.at[pl.ds(zoff, RPT)],
                        acc.at[pl.ds(zoff, RPT)])
        plsc.subcore_barrier()

        def wait_fetch(b, chunk):
            pltpu.make_async_copy(v_hbm.at[sidx.at[chunk]], rows[b],
                                  gsem[b]).wait()
            pltpu.make_async_copy(dst_hbm.at[wid, chunk], didxb[b],
                                  dsem[b]).wait()

        def fetch(chunk, b):
            pltpu.async_copy(v_hbm.at[sidx.at[chunk]], rows[b], gsem[b])
            pltpu.async_copy(dst_hbm.at[wid, chunk], didxb[b], dsem[b])

        for b in range(PD):
            fetch(jnp.int32(b), b)

        def group(g, carry):
            for b in range(NBUF):
                i = g * jnp.int32(NBUF) + jnp.int32(b)
                j = i + jnp.int32(PD)
                bj = (b + PD) % NBUF
                wait_fetch(b, i)
                if b < PD:
                    fetch(j, bj)
                else:
                    @pl.when(g < jnp.int32(NG - 1))
                    def _():
                        fetch(j, bj)
                pltpu.sync_copy(rows[b], acc.at[didxb[b]], add=True)
            return carry

        lax.fori_loop(jnp.int32(0), jnp.int32(NG), group, jnp.int32(0))
        plsc.subcore_barrier()
        pltpu.sync_copy(acc.at[pl.ds(zoff, RPT)],
                        out_hbm.at[c, pl.ds(zoff, RPT)])

    return agg


def _tc_mm1_body(x_ref, w1_ref, o_ref):
    o_ref[...] = jnp.dot(x_ref[...], w1_ref[...],
                         preferred_element_type=jnp.float32)


def _tc_scale_body(degp_ref, u1_ref, v1_ref, rsqi_ref, rsqo_ref):
    dp = degp_ref[...]
    deg_out = dp[0, 0, :N] + dp[1, 0, :N]
    deg_in = dp[0, 1, :N] + dp[1, 1, :N]
    rsq_out = lax.rsqrt(jnp.maximum(deg_out, 1.0))[:, None]
    rsq_in = lax.rsqrt(jnp.maximum(deg_in, 1.0))[:, None]
    v1_ref[...] = u1_ref[...] * rsq_out
    rsqi_ref[...] = rsq_in
    rsqo_ref[...] = rsq_out


def _tc_layer2_body(aggp_ref, rsqi_ref, rsqo_ref, b1_ref, w2p_ref, v2_ref):
    a = aggp_ref[0, :N, :] + aggp_ref[1, :N, :]
    h1 = jnp.maximum(a * rsqi_ref[...] + b1_ref[...][None, :], 0.0)
    v2_ref[...] = jnp.dot(h1 * rsqo_ref[...], w2p_ref[...],
                          preferred_element_type=jnp.float32)


def _tc_head_body(aggp_ref, rsqi_ref, b2_ref, wa_ref, wm1_ref, bm1_ref,
                  wm2_ref, bm2_ref, wm3_ref, bm3_ref, o_ref):
    a = aggp_ref[0, :N, :D2] + aggp_ref[1, :N, :D2]
    h2 = jnp.maximum(a * rsqi_ref[...] + b2_ref[...][None, :], 0.0)
    m = jnp.mean(h2, axis=0, keepdims=True)
    tg = jnp.tanh(jnp.dot(m, wa_ref[...],
                          preferred_element_type=jnp.float32))
    scores = jax.nn.sigmoid(jnp.sum(h2 * tg, axis=1, keepdims=True))
    rep = jnp.sum(h2 * scores, axis=0, keepdims=True)
    z = jnp.maximum(jnp.dot(rep, wm1_ref[...],
                            preferred_element_type=jnp.float32)
                    + bm1_ref[...][None, :], 0.0)
    z = jnp.maximum(jnp.dot(z, wm2_ref[...],
                            preferred_element_type=jnp.float32)
                    + bm2_ref[...][None, :], 0.0)
    o_ref[...] = (jnp.dot(z, wm3_ref[...],
                          preferred_element_type=jnp.float32)
                  + bm3_ref[...][None, :])


def _tc_call(body, out_shape, *args):
    return pl.pallas_call(body, out_shape=out_shape)(*args)


def kernel(x, edge_index, W1, b1, W2, b2, Wa, Wm1, bm1, Wm2, bm2, Wm3, bm3):
    srcf = edge_index[0].astype(jnp.int32)
    dstf = edge_index[1].astype(jnp.int32)
    src = srcf.reshape(NW, NCHUNK, CH)
    dst = dstf.reshape(NW, NCHUNK, CH)
    zdeg = jnp.zeros((NPAD,), jnp.float32)
    z1 = jnp.zeros((NPAD, D1), jnp.float32)
    z2 = jnp.zeros((NPAD, D2P), jnp.float32)
    W2p = jnp.pad(W2, ((0, 0), (0, D2P - D2)))

    degp = _get_sc_degrees()(srcf, dstf, zdeg)
    u1 = _tc_call(_tc_mm1_body,
                  jax.ShapeDtypeStruct((N, D1), jnp.float32), x, W1)
    v1, rsq_in, rsq_out = _tc_call(
        _tc_scale_body,
        (jax.ShapeDtypeStruct((N, D1), jnp.float32),
         jax.ShapeDtypeStruct((N, 1), jnp.float32),
         jax.ShapeDtypeStruct((N, 1), jnp.float32)),
        degp, u1)
    agg1p = _make_sc_aggregate(D1)(v1, src, dst, z1)
    v2 = _tc_call(_tc_layer2_body,
                  jax.ShapeDtypeStruct((N, D2P), jnp.float32),
                  agg1p, rsq_in, rsq_out, b1, W2p)
    agg2p = _make_sc_aggregate(D2P)(v2, src, dst, z2)
    out = _tc_call(_tc_head_body,
                   jax.ShapeDtypeStruct((1, 1), jnp.float32),
                   agg2p, rsq_in, b2, Wa, Wm1, bm1, Wm2, bm2, Wm3, bm3)
    return out
